# baseline (speedup 1.0000x reference)
import functools

import jax
import jax.numpy as jnp
from jax.experimental import pallas as pl
import jax.experimental.pallas.tpu as pltpu

BATCH = 2048
DIM = 2048
NUM_EXPERTS = 8
ATTN_GAIN_MEAN = 1.0 + 32.0 / 2048.0
KT = 512
JT = 256


def _fused_kernel(x_ref, gw_ref, gb_ref, ex_ref, out_ref, xbf_ref, w_ref):
    j = pl.program_id(0)
    k = pl.program_id(1)

    @pl.when(jnp.logical_and(j == 0, k == 0))
    def _init():
        x = x_ref[...]
        mu = jnp.mean(x, axis=1, keepdims=True)
        xc = x - mu
        std = jnp.sqrt(jnp.mean(xc * xc, axis=1, keepdims=True)) + 1e-6
        xbf_ref[...] = (xc / std).astype(jnp.bfloat16)
        logits = gw_ref[0:1, :] + ATTN_GAIN_MEAN * gw_ref[1:2, :] + gb_ref[...]
        m = jnp.max(logits, axis=1, keepdims=True)
        e = jnp.exp(logits - m)
        w_ref[...] = e / jnp.sum(e, axis=1, keepdims=True)

    ex = ex_ref[...]
    comb = w_ref[0, 0] * ex[0]
    for e_idx in range(1, NUM_EXPERTS):
        comb = comb + w_ref[0, e_idx] * ex[e_idx]

    xk = xbf_ref[:, pl.ds(k * KT, KT)]
    acc = jnp.dot(xk, comb.astype(jnp.bfloat16),
                  preferred_element_type=jnp.float32)

    @pl.when(k == 0)
    def _first():
        out_ref[...] = acc

    @pl.when(k > 0)
    def _rest():
        out_ref[...] += acc


@functools.partial(jax.jit, static_argnames=())
def kernel(query_embedding, gate_W, gate_b, experts):
    nj = DIM // JT
    nk = DIM // KT
    gate_b2 = gate_b.reshape(1, NUM_EXPERTS)
    return pl.pallas_call(
        _fused_kernel,
        grid=(nj, nk),
        in_specs=[
            pl.BlockSpec((BATCH, DIM), lambda j, k: (0, 0)),
            pl.BlockSpec((12, NUM_EXPERTS), lambda j, k: (0, 0)),
            pl.BlockSpec((1, NUM_EXPERTS), lambda j, k: (0, 0)),
            pl.BlockSpec((NUM_EXPERTS, KT, JT), lambda j, k: (0, k, j)),
        ],
        out_specs=pl.BlockSpec((BATCH, JT), lambda j, k: (0, j)),
        out_shape=jax.ShapeDtypeStruct((BATCH, DIM), jnp.float32),
        scratch_shapes=[pltpu.VMEM((BATCH, DIM), jnp.bfloat16),
                        pltpu.VMEM((1, NUM_EXPERTS), jnp.float32)],
    )(query_embedding, gate_W, gate_b2, experts)


# speedup vs baseline: 1.1010x; 1.1010x over previous
import jax
import jax.numpy as jnp
from jax.experimental import pallas as pl
import jax.experimental.pallas.tpu as pltpu

BATCH = 2048
DIM = 2048
NUM_EXPERTS = 8
ATTN_GAIN_MEAN = 1.0 + 32.0 / 2048.0
KT = 256
JT = 1024


def _fused_kernel(x_ref, gw_ref, gb_ref, ex_ref, out_ref,
                  mu_ref, rstd_ref, w_ref):
    j = pl.program_id(0)
    k = pl.program_id(1)

    @pl.when(jnp.logical_and(j == 0, k == 0))
    def _init():
        x = x_ref[...]
        mu = jnp.mean(x, axis=1, keepdims=True)
        xc = x - mu
        std = jnp.sqrt(jnp.mean(xc * xc, axis=1, keepdims=True)) + 1e-6
        mu_ref[...] = mu
        rstd_ref[...] = 1.0 / std
        logits = gw_ref[0:1, :] + ATTN_GAIN_MEAN * gw_ref[1:2, :] + gb_ref[...]
        m = jnp.max(logits, axis=1, keepdims=True)
        e = jnp.exp(logits - m)
        w_ref[...] = e / jnp.sum(e, axis=1, keepdims=True)

    ex = ex_ref[...]
    comb = w_ref[0, 0] * ex[0]
    for e_idx in range(1, NUM_EXPERTS):
        comb = comb + w_ref[0, e_idx] * ex[e_idx]

    xk = (x_ref[:, pl.ds(k * KT, KT)] - mu_ref[...]) * rstd_ref[...]
    acc = jnp.dot(xk.astype(jnp.bfloat16), comb.astype(jnp.bfloat16),
                  preferred_element_type=jnp.float32)

    @pl.when(k == 0)
    def _first():
        out_ref[...] = acc

    @pl.when(k > 0)
    def _rest():
        out_ref[...] += acc


@jax.jit
def kernel(query_embedding, gate_W, gate_b, experts):
    nj = DIM // JT
    nk = DIM // KT
    gate_b2 = gate_b.reshape(1, NUM_EXPERTS)
    return pl.pallas_call(
        _fused_kernel,
        grid=(nj, nk),
        in_specs=[
            pl.BlockSpec((BATCH, DIM), lambda j, k: (0, 0)),
            pl.BlockSpec((12, NUM_EXPERTS), lambda j, k: (0, 0)),
            pl.BlockSpec((1, NUM_EXPERTS), lambda j, k: (0, 0)),
            pl.BlockSpec((NUM_EXPERTS, KT, JT), lambda j, k: (0, k, j)),
        ],
        out_specs=pl.BlockSpec((BATCH, JT), lambda j, k: (0, j)),
        out_shape=jax.ShapeDtypeStruct((BATCH, DIM), jnp.float32),
        scratch_shapes=[pltpu.VMEM((BATCH, 1), jnp.float32),
                        pltpu.VMEM((BATCH, 1), jnp.float32),
                        pltpu.VMEM((1, NUM_EXPERTS), jnp.float32)],
    )(query_embedding, gate_W, gate_b2, experts)
